# baseline (speedup 1.0000x reference)
import functools

import jax
import jax.numpy as jnp
from jax import lax
from jax.experimental import pallas as pl
from jax.experimental.pallas import tpu as pltpu
from jax.experimental.pallas import tpu_sc as plsc

B = 16
N = 20000
C = 80
M = 1000
S = 8000

L = 16
NSTEP = S // L
CAP = 1008
CHUNK = 112
NCHUNK = CAP // CHUNK


def _body(boxes_hbm, scores_hbm, b_hbm, x_hbm, l_hbm,
          num_out, boxes_out, scores_out, classes_out,
          bbuf, xbuf, lbuf, posbuf, idxbuf, e7buf, boxg, scoreg, sout,
          clsout, cntbuf, sem):
    cid = lax.axis_index("c")
    b = lax.axis_index("s")
    lanes = lax.iota(jnp.int32, L)
    cid_splat = lax.broadcast(cid, (L,))
    b_splat = lax.broadcast(b, (L,))

    pltpu.sync_copy(b_hbm, bbuf)
    pltpu.sync_copy(x_hbm, xbuf)
    pltpu.sync_copy(l_hbm, lbuf)

    cntbuf[...] = jnp.zeros((L,), jnp.int32)

    def scan_step(j, carry):
        bv = bbuf[pl.ds(j * L, L)]
        m = bv == b_splat
        pref = plsc.cumsum(m.astype(jnp.int32))
        cnt_cur = cntbuf[...]
        dest = jnp.where(m, cnt_cur + pref - 1, jnp.int32(S + L) + lanes)
        plsc.store_scatter(posbuf, [dest], lanes + j * L)
        cntbuf[...] = cnt_cur + plsc.all_reduce_population_count(m)
        return carry

    lax.fori_loop(0, NSTEP, scan_step, 0)
    cnt_splat = cntbuf[...]
    cnt_m = jnp.minimum(cnt_splat, M)

    def idx_outer(t, _):
        def idx_inner(u, _):
            k = t * (CHUNK // L) + u
            slot = lanes + k * L
            valid = slot < cnt_m
            posv = plsc.load_gather(posbuf, [jnp.where(valid, slot, 0)])
            posv = jnp.where(valid, posv, 0)
            xv = plsc.load_gather(xbuf, [posv])
            lv = plsc.load_gather(lbuf, [posv])
            row = b_splat * N + xv
            boxrow = jnp.where(valid, row, B * N)
            e = row * C + lv
            erow8 = jnp.where(valid, lax.shift_right_logical(e, 3), 0)
            idxbuf[t, pl.ds(u * L, L)] = jnp.where(cid_splat == 0, boxrow,
                                                   erow8)
            e7buf[pl.ds(k * L, L)] = jnp.bitwise_and(e, 7)
            clsout[pl.ds(k * L, L)] = jnp.where(valid, lv, 0)
            return 0

        lax.fori_loop(0, CHUNK // L, idx_inner, 0)
        return 0

    lax.fori_loop(0, NCHUNK, idx_outer, 0)

    @pl.when(cid == 0)
    def _boxes():
        copies = []
        for t in range(NCHUNK):
            dst = boxg.at[pl.ds(t * CHUNK, CHUNK)]
            copies.append(
                pltpu.async_copy(boxes_hbm.at[idxbuf.at[t]], dst, sem))
        for cdesc in copies:
            cdesc.wait()
        pltpu.sync_copy(boxg, boxes_out.at[b])

    @pl.when(cid == 1)
    def _scores():
        copies = []
        for t in range(NCHUNK):
            dst = scoreg.at[pl.ds(t * CHUNK, CHUNK)]
            copies.append(
                pltpu.async_copy(scores_hbm.at[idxbuf.at[t]], dst, sem))
        for cdesc in copies:
            cdesc.wait()

        def ext_step(k, _):
            slot = lanes + k * L
            valid = slot < cnt_m
            ev7 = e7buf[pl.ds(k * L, L)]
            sv = plsc.load_gather(scoreg, [slot, ev7])
            sout[pl.ds(k * L, L)] = jnp.where(valid, sv, jnp.float32(0))
            return 0

        lax.fori_loop(0, CAP // L, ext_step, 0)
        pltpu.sync_copy(sout, scores_out.at[b])
        pltpu.sync_copy(clsout, classes_out.at[b])
        pltpu.sync_copy(cntbuf, num_out.at[b])


@functools.partial(
    pl.kernel,
    out_type=(
        jax.ShapeDtypeStruct((B, L), jnp.int32),
        jax.ShapeDtypeStruct((B, CAP, 8), jnp.float32),
        jax.ShapeDtypeStruct((B, CAP), jnp.float32),
        jax.ShapeDtypeStruct((B, CAP), jnp.int32),
    ),
    mesh=plsc.VectorSubcoreMesh(
        core_axis_name="c", subcore_axis_name="s", num_cores=2,
        num_subcores=16),
    compiler_params=pltpu.CompilerParams(
        needs_layout_passes=False, use_tc_tiling_on_sc=False),
    scratch_types=(
        pltpu.VMEM((S,), jnp.int32),
        pltpu.VMEM((S,), jnp.int32),
        pltpu.VMEM((S,), jnp.int32),
        pltpu.VMEM((S + 2 * L,), jnp.int32),
        pltpu.VMEM((NCHUNK, CHUNK), jnp.int32),
        pltpu.VMEM((CAP,), jnp.int32),
        pltpu.VMEM((CAP, 8), jnp.float32),
        pltpu.VMEM((CAP, 8), jnp.float32),
        pltpu.VMEM((CAP,), jnp.float32),
        pltpu.VMEM((CAP,), jnp.int32),
        pltpu.VMEM((L,), jnp.int32),
        pltpu.SemaphoreType.DMA,
    ),
)
def _sc_pick(boxes_hbm, scores_hbm, b_hbm, x_hbm, l_hbm,
             num_out, boxes_out, scores_out, classes_out,
             *scratch):
    _body(boxes_hbm, scores_hbm, b_hbm, x_hbm, l_hbm,
          num_out, boxes_out, scores_out, classes_out, *scratch)


@jax.jit
def kernel(pred_boxes, pred_scores, selected_indexes):
    sel = selected_indexes.astype(jnp.int32)
    bi = sel[:, 0]
    li = sel[:, 1]
    xi = sel[:, 2]
    boxes8 = jnp.pad(pred_boxes.reshape(B * N, 4), ((0, L), (0, 4)))
    scores8 = pred_scores.reshape(B * N * C // 8, 8)
    num, ob, osc, ocl = _sc_pick(boxes8, scores8, bi, xi, li)
    return num[:, :1], ob[:, :M, :4], osc[:, :M], ocl[:, :M]
